# baseline (speedup 1.0000x reference)
import functools

import jax
import jax.numpy as jnp
from jax import lax
from jax.experimental import pallas as pl
from jax.experimental.pallas import tpu as pltpu
from jax.experimental.pallas import tpu_sc as plsc

V = 1000000
NEW = 1024
D = 64
DP = 128
NC = 2
NS = 16
NW = NC * NS

BTILE = 128
LQ = 4


def _embed_kernel(B, L, orig_hbm, new_hbm, idsT_hbm, out_hbm,
                  ids_v, oidx_v, new_v, buf_a, buf_b, tbuf_a, tbuf_b,
                  gsem_a, gsem_b, wsem_a, wsem_b):
    n_l = L // LQ
    wid = lax.axis_index("s") * NC + lax.axis_index("c")
    bt = wid // LQ
    l0 = (wid % LQ) * n_l
    b0 = bt * BTILE

    pltpu.sync_copy(new_hbm, new_v)
    pltpu.sync_copy(idsT_hbm.at[pl.ds(l0, n_l), pl.ds(b0, BTILE)], ids_v)

    def idx_body(li, _):
        for k in range(BTILE // 16):
            v16 = ids_v[li, pl.ds(k * 16, 16)]
            oidx_v[li, pl.ds(k * 16, 16)] = jnp.minimum(v16, V - 1)
        return _
    lax.fori_loop(0, n_l, idx_body, None)

    lane = lax.iota(jnp.int32, 16)
    bufs = (buf_a, buf_b)
    tbufs = (tbuf_a, tbuf_b)
    gsems = (gsem_a, gsem_b)
    wsems = (wsem_a, wsem_b)
    nbuf = 2

    def fire(li, b):
        pltpu.async_copy(orig_hbm.at[oidx_v.at[li]], bufs[b], gsems[b])

    def out_slice(li):
        return out_hbm.at[l0 + li, :, bt, :, :]

    def process(li, b):
        buf = bufs[b]
        tbuf = tbufs[b]
        for k in range(BTILE // 16):
            pos = k * 16
            ids16 = ids_v[li, pl.ds(pos, 16)]
            m16 = ids16 >= V

            @pl.when(jnp.any(m16))
            def _():
                nidx16 = jnp.maximum(ids16 - V, 0)
                rows16 = pos + lane
                for c in range(D):
                    col16 = jnp.full((16,), c, jnp.int32)
                    q = plsc.load_gather(new_v, [nidx16, col16], mask=m16)
                    plsc.store_scatter(buf, [rows16, col16], q, mask=m16)

        def tr_body(fl):
            d = fl // 8
            k = fl % 8
            q = plsc.load_gather(buf, [k * 16 + lane, jnp.full((16,), d, jnp.int32)])
            tbuf[d // 8, d % 8, pl.ds(k * 16, 16)] = q

        plsc.parallel_loop(0, D * 8, 1, unroll=32)(tr_body)
        pltpu.async_copy(tbuf, out_slice(li), wsems[b])

    for b in range(nbuf):
        fire(b, b)

    def pair_body(p, _):
        for b in range(nbuf):
            li = p * nbuf + b
            pltpu.make_async_copy(
                orig_hbm.at[oidx_v.at[li]], bufs[b], gsems[b]).wait()

            @pl.when(p > 0)
            def _():
                pltpu.make_async_copy(
                    tbufs[b], out_slice(li - nbuf), wsems[b]).wait()

            process(li, b)

            @pl.when(p < (n_l // nbuf) - 1)
            def _():
                fire(li + nbuf, b)
        return _

    lax.fori_loop(0, n_l // nbuf, pair_body, None)
    for b in range(nbuf):
        pltpu.make_async_copy(
            tbufs[b], out_slice(n_l - nbuf + b), wsems[b]).wait()


def kernel(orig_weight, new_weight, input_ids):
    B, L = input_ids.shape
    idsT = input_ids.T.astype(jnp.int32)
    orig_p = jnp.pad(orig_weight, ((0, 0), (0, DP - D)))

    mesh = plsc.VectorSubcoreMesh(core_axis_name="c", subcore_axis_name="s")
    run = pl.kernel(
        functools.partial(_embed_kernel, B, L),
        out_type=jax.ShapeDtypeStruct((L, D // 8, B // BTILE, 8, BTILE),
                                      jnp.float32),
        mesh=mesh,
        compiler_params=pltpu.CompilerParams(
            use_tc_tiling_on_sc=False, needs_layout_passes=False),
        scratch_types=[
            pltpu.VMEM((L // LQ, BTILE), jnp.int32),
            pltpu.VMEM((L // LQ, BTILE), jnp.int32),
            pltpu.VMEM((NEW, D), jnp.float32),
            pltpu.VMEM((BTILE, DP), jnp.float32),
            pltpu.VMEM((BTILE, DP), jnp.float32),
            pltpu.VMEM((D // 8, 8, BTILE), jnp.float32),
            pltpu.VMEM((D // 8, 8, BTILE), jnp.float32),
            pltpu.SemaphoreType.DMA,
            pltpu.SemaphoreType.DMA,
            pltpu.SemaphoreType.DMA,
            pltpu.SemaphoreType.DMA,
        ],
    )
    out5 = run(orig_p, new_weight, idsT)
    return out5.transpose(2, 4, 0, 1, 3).reshape(B, L, D)


# speedup vs baseline: 1.0306x; 1.0306x over previous
import functools

import jax
import jax.numpy as jnp
from jax import lax
from jax.experimental import pallas as pl
from jax.experimental.pallas import tpu as pltpu
from jax.experimental.pallas import tpu_sc as plsc

V = 1000000
NEW = 1024
D = 64
DP = 128
NC = 2
NS = 16
NW = NC * NS

BTILE = 128
LQ = 4


def _embed_kernel(B, L, orig_hbm, new_hbm, idsT_hbm, out_hbm,
                  ids_v, oidx_v, new_v, buf_a, buf_b, tbuf_a, tbuf_b,
                  gsem_a, gsem_b, wsem_a, wsem_b):
    n_l = L // LQ
    wid = lax.axis_index("s") * NC + lax.axis_index("c")
    bt = wid // LQ
    l0 = (wid % LQ) * n_l
    b0 = bt * BTILE

    pltpu.sync_copy(new_hbm, new_v)
    pltpu.sync_copy(idsT_hbm.at[pl.ds(l0, n_l), pl.ds(b0, BTILE)], ids_v)

    def idx_body(li, _):
        for k in range(BTILE // 16):
            v16 = ids_v[li, pl.ds(k * 16, 16)]
            oidx_v[li, pl.ds(k * 16, 16)] = jnp.minimum(v16, V - 1)
        return _
    lax.fori_loop(0, n_l, idx_body, None)

    lane = lax.iota(jnp.int32, 16)
    bufs = (buf_a, buf_b)
    tbufs = (tbuf_a, tbuf_b)
    gsems = (gsem_a, gsem_b)
    wsems = (wsem_a, wsem_b)
    nbuf = 2

    def fire(li, b):
        pltpu.async_copy(orig_hbm.at[oidx_v.at[li]], bufs[b], gsems[b])

    def out_slice(li):
        return out_hbm.at[l0 + li, :, bt, :, :]

    def process(li, b):
        buf = bufs[b]
        tbuf = tbufs[b]
        for k in range(BTILE // 16):
            pos = k * 16
            ids16 = ids_v[li, pl.ds(pos, 16)]
            m16 = ids16 >= V

            @pl.when(jnp.any(m16))
            def _():
                nidx16 = jnp.maximum(ids16 - V, 0)
                rows16 = pos + lane
                for c in range(D):
                    col16 = jnp.full((16,), c, jnp.int32)
                    q = plsc.load_gather(new_v, [nidx16, col16], mask=m16)
                    plsc.store_scatter(buf, [rows16, col16], q, mask=m16)

        def tr_body(d):
            col16 = jnp.full((16,), d, jnp.int32)
            for k in range(BTILE // 16):
                q = plsc.load_gather(buf, [k * 16 + lane, col16])
                tbuf[d // 8, d % 8, pl.ds(k * 16, 16)] = q

        plsc.parallel_loop(0, D, 1, unroll=8)(tr_body)
        pltpu.async_copy(tbuf, out_slice(li), wsems[b])

    for b in range(nbuf):
        fire(b, b)

    def pair_body(p, _):
        for b in range(nbuf):
            li = p * nbuf + b
            pltpu.make_async_copy(
                orig_hbm.at[oidx_v.at[li]], bufs[b], gsems[b]).wait()

            @pl.when(p > 0)
            def _():
                pltpu.make_async_copy(
                    tbufs[b], out_slice(li - nbuf), wsems[b]).wait()

            process(li, b)

            @pl.when(p < (n_l // nbuf) - 1)
            def _():
                fire(li + nbuf, b)
        return _

    lax.fori_loop(0, n_l // nbuf, pair_body, None)
    for b in range(nbuf):
        pltpu.make_async_copy(
            tbufs[b], out_slice(n_l - nbuf + b), wsems[b]).wait()


def kernel(orig_weight, new_weight, input_ids):
    B, L = input_ids.shape
    idsT = input_ids.T.astype(jnp.int32)
    orig_p = jnp.concatenate(
        [orig_weight, jnp.zeros((V, DP - D), jnp.float32)], axis=1)

    mesh = plsc.VectorSubcoreMesh(core_axis_name="c", subcore_axis_name="s")
    run = pl.kernel(
        functools.partial(_embed_kernel, B, L),
        out_type=jax.ShapeDtypeStruct((L, D // 8, B // BTILE, 8, BTILE),
                                      jnp.float32),
        mesh=mesh,
        compiler_params=pltpu.CompilerParams(
            use_tc_tiling_on_sc=False, needs_layout_passes=False),
        scratch_types=[
            pltpu.VMEM((L // LQ, BTILE), jnp.int32),
            pltpu.VMEM((L // LQ, BTILE), jnp.int32),
            pltpu.VMEM((NEW, D), jnp.float32),
            pltpu.VMEM((BTILE, DP), jnp.float32),
            pltpu.VMEM((BTILE, DP), jnp.float32),
            pltpu.VMEM((D // 8, 8, BTILE), jnp.float32),
            pltpu.VMEM((D // 8, 8, BTILE), jnp.float32),
            pltpu.SemaphoreType.DMA,
            pltpu.SemaphoreType.DMA,
            pltpu.SemaphoreType.DMA,
            pltpu.SemaphoreType.DMA,
        ],
    )
    out5 = run(orig_p, new_weight, idsT)
    return out5.transpose(2, 4, 0, 1, 3).reshape(B, L, D)


# speedup vs baseline: 1.0321x; 1.0014x over previous
import functools

import jax
import jax.numpy as jnp
from jax import lax
from jax.experimental import pallas as pl
from jax.experimental.pallas import tpu as pltpu
from jax.experimental.pallas import tpu_sc as plsc

V = 1000000
NEW = 1024
D = 64
DP = 128
NC = 2
NS = 16
NW = NC * NS

BTILE = 128
LQ = 4


def _embed_kernel(B, L, orig_hbm, new_hbm, idsT_hbm, out_hbm,
                  ids_v, oidx_v, new_v, buf_a, buf_b, tbuf_a, tbuf_b,
                  gsem_a, gsem_b, wsem_a, wsem_b):
    n_l = L // LQ
    wid = lax.axis_index("s") * NC + lax.axis_index("c")
    bt = wid // LQ
    l0 = (wid % LQ) * n_l
    b0 = bt * BTILE

    pltpu.sync_copy(new_hbm, new_v)
    pltpu.sync_copy(idsT_hbm.at[pl.ds(l0, n_l), pl.ds(b0, BTILE)], ids_v)

    def idx_body(li, _):
        for k in range(BTILE // 16):
            v16 = ids_v[li, pl.ds(k * 16, 16)]
            oidx_v[li, pl.ds(k * 16, 16)] = jnp.minimum(v16, V - 1)
        return _
    lax.fori_loop(0, n_l, idx_body, None)

    lane = lax.iota(jnp.int32, 16)
    bufs = (buf_a, buf_b)
    tbufs = (tbuf_a, tbuf_b)
    gsems = (gsem_a, gsem_b)
    wsems = (wsem_a, wsem_b)
    nbuf = 2

    def fire(li, b):
        pltpu.async_copy(orig_hbm.at[oidx_v.at[li]], bufs[b], gsems[b])

    def out_slice(li):
        return out_hbm.at[l0 + li, :, bt, :, :]

    def process(li, b):
        buf = bufs[b]
        tbuf = tbufs[b]
        for k in range(BTILE // 16):
            pos = k * 16
            ids16 = ids_v[li, pl.ds(pos, 16)]
            m16 = ids16 >= V

            @pl.when(jnp.any(m16))
            def _():
                nidx16 = jnp.maximum(ids16 - V, 0)
                rows16 = pos + lane
                for c in range(D):
                    col16 = jnp.full((16,), c, jnp.int32)
                    q = plsc.load_gather(new_v, [nidx16, col16], mask=m16)
                    plsc.store_scatter(buf, [rows16, col16], q, mask=m16)

        dts = [((c * 16 + lane) // 8, (c * 16 + lane) % 8)
               for c in range(D // 16)]

        def tr_body(bs):
            bs16 = jnp.full((16,), bs, jnp.int32)
            for c in range(D // 16):
                q = buf[bs, pl.ds(c * 16, 16)]
                plsc.store_scatter(tbuf, [dts[c][0], dts[c][1], bs16], q)

        plsc.parallel_loop(0, BTILE, 1, unroll=8)(tr_body)
        pltpu.async_copy(tbuf, out_slice(li), wsems[b])

    for b in range(nbuf):
        fire(b, b)

    def pair_body(p, _):
        for b in range(nbuf):
            li = p * nbuf + b
            pltpu.make_async_copy(
                orig_hbm.at[oidx_v.at[li]], bufs[b], gsems[b]).wait()

            @pl.when(p > 0)
            def _():
                pltpu.make_async_copy(
                    tbufs[b], out_slice(li - nbuf), wsems[b]).wait()

            process(li, b)

            @pl.when(p < (n_l // nbuf) - 1)
            def _():
                fire(li + nbuf, b)
        return _

    lax.fori_loop(0, n_l // nbuf, pair_body, None)
    for b in range(nbuf):
        pltpu.make_async_copy(
            tbufs[b], out_slice(n_l - nbuf + b), wsems[b]).wait()


def kernel(orig_weight, new_weight, input_ids):
    B, L = input_ids.shape
    idsT = input_ids.T.astype(jnp.int32)
    orig_p = jnp.concatenate(
        [orig_weight, jnp.zeros((V, DP - D), jnp.float32)], axis=1)

    mesh = plsc.VectorSubcoreMesh(core_axis_name="c", subcore_axis_name="s")
    run = pl.kernel(
        functools.partial(_embed_kernel, B, L),
        out_type=jax.ShapeDtypeStruct((L, D // 8, B // BTILE, 8, BTILE),
                                      jnp.float32),
        mesh=mesh,
        compiler_params=pltpu.CompilerParams(
            use_tc_tiling_on_sc=False, needs_layout_passes=False),
        scratch_types=[
            pltpu.VMEM((L // LQ, BTILE), jnp.int32),
            pltpu.VMEM((L // LQ, BTILE), jnp.int32),
            pltpu.VMEM((NEW, D), jnp.float32),
            pltpu.VMEM((BTILE, DP), jnp.float32),
            pltpu.VMEM((BTILE, DP), jnp.float32),
            pltpu.VMEM((D // 8, 8, BTILE), jnp.float32),
            pltpu.VMEM((D // 8, 8, BTILE), jnp.float32),
            pltpu.SemaphoreType.DMA,
            pltpu.SemaphoreType.DMA,
            pltpu.SemaphoreType.DMA,
            pltpu.SemaphoreType.DMA,
        ],
    )
    out5 = run(orig_p, new_weight, idsT)
    return out5.transpose(2, 4, 0, 1, 3).reshape(B, L, D)


# speedup vs baseline: 1.0641x; 1.0310x over previous
import functools

import jax
import jax.numpy as jnp
from jax import lax
from jax.experimental import pallas as pl
from jax.experimental.pallas import tpu as pltpu
from jax.experimental.pallas import tpu_sc as plsc

V = 1000000
NEW = 1024
D = 64
DP = 128
NC = 2
NS = 16
NW = NC * NS

BTILE = 128
LQ = 4


def _embed_kernel(B, L, orig_hbm, new_hbm, idsT_hbm, out_hbm,
                  ids_v, oidx_v, flags_v, new_v, buf_a, buf_b, tbuf_a, tbuf_b,
                  gsem_a, gsem_b, wsem_a, wsem_b):
    n_l = L // LQ
    wid = lax.axis_index("s") * NC + lax.axis_index("c")
    bt = wid // LQ
    l0 = (wid % LQ) * n_l
    b0 = bt * BTILE

    pltpu.sync_copy(new_hbm, new_v)
    pltpu.sync_copy(idsT_hbm.at[pl.ds(l0, n_l), pl.ds(b0, BTILE)], ids_v)

    def idx_body(li, _):
        acc = jnp.zeros((16,), jnp.int32)
        for k in range(BTILE // 16):
            v16 = ids_v[li, pl.ds(k * 16, 16)]
            oidx_v[li, pl.ds(k * 16, 16)] = jnp.minimum(v16, V - 1)
            acc = jnp.maximum(acc, v16)
        flags_v[li, :] = acc
        return _
    lax.fori_loop(0, n_l, idx_body, None)

    lane = lax.iota(jnp.int32, 16)
    bufs = (buf_a, buf_b)
    tbufs = (tbuf_a, tbuf_b)
    gsems = (gsem_a, gsem_b)
    wsems = (wsem_a, wsem_b)
    nbuf = 2

    def fire(li, b):
        pltpu.async_copy(orig_hbm.at[oidx_v.at[li]], bufs[b], gsems[b])

    def out_slice(li):
        return out_hbm.at[l0 + li, :, bt, :, :]

    def process(li, b):
        buf = bufs[b]
        tbuf = tbufs[b]
        @pl.when(jnp.any(flags_v[li, :] >= V))
        def _():
            for k in range(BTILE // 16):
                pos = k * 16
                ids16 = ids_v[li, pl.ds(pos, 16)]
                m16 = ids16 >= V

                @pl.when(jnp.any(m16))
                def _():
                    nidx16 = jnp.maximum(ids16 - V, 0)
                    rows16 = pos + lane
                    for c in range(D):
                        col16 = jnp.full((16,), c, jnp.int32)
                        q = plsc.load_gather(new_v, [nidx16, col16], mask=m16)
                        plsc.store_scatter(buf, [rows16, col16], q, mask=m16)

        dts = [((c * 16 + lane) // 8, (c * 16 + lane) % 8)
               for c in range(D // 16)]

        def tr_body(bs):
            bs16 = jnp.full((16,), bs, jnp.int32)
            for c in range(D // 16):
                q = buf[bs, pl.ds(c * 16, 16)]
                plsc.store_scatter(tbuf, [dts[c][0], dts[c][1], bs16], q)

        plsc.parallel_loop(0, BTILE, 1, unroll=8)(tr_body)
        pltpu.async_copy(tbuf, out_slice(li), wsems[b])

    for b in range(nbuf):
        fire(b, b)

    def pair_body(p, _):
        for b in range(nbuf):
            li = p * nbuf + b
            pltpu.make_async_copy(
                orig_hbm.at[oidx_v.at[li]], bufs[b], gsems[b]).wait()

            @pl.when(p > 0)
            def _():
                pltpu.make_async_copy(
                    tbufs[b], out_slice(li - nbuf), wsems[b]).wait()

            process(li, b)

            @pl.when(p < (n_l // nbuf) - 1)
            def _():
                fire(li + nbuf, b)
        return _

    lax.fori_loop(0, n_l // nbuf, pair_body, None)
    for b in range(nbuf):
        pltpu.make_async_copy(
            tbufs[b], out_slice(n_l - nbuf + b), wsems[b]).wait()


def kernel(orig_weight, new_weight, input_ids):
    B, L = input_ids.shape
    idsT = input_ids.T.astype(jnp.int32)
    orig_p = jnp.concatenate(
        [orig_weight, jnp.zeros((V, DP - D), jnp.float32)], axis=1)

    mesh = plsc.VectorSubcoreMesh(core_axis_name="c", subcore_axis_name="s")
    run = pl.kernel(
        functools.partial(_embed_kernel, B, L),
        out_type=jax.ShapeDtypeStruct((L, D // 8, B // BTILE, 8, BTILE),
                                      jnp.float32),
        mesh=mesh,
        compiler_params=pltpu.CompilerParams(
            use_tc_tiling_on_sc=False, needs_layout_passes=False),
        scratch_types=[
            pltpu.VMEM((L // LQ, BTILE), jnp.int32),
            pltpu.VMEM((L // LQ, BTILE), jnp.int32),
            pltpu.VMEM((L // LQ, 16), jnp.int32),
            pltpu.VMEM((NEW, D), jnp.float32),
            pltpu.VMEM((BTILE, DP), jnp.float32),
            pltpu.VMEM((BTILE, DP), jnp.float32),
            pltpu.VMEM((D // 8, 8, BTILE), jnp.float32),
            pltpu.VMEM((D // 8, 8, BTILE), jnp.float32),
            pltpu.SemaphoreType.DMA,
            pltpu.SemaphoreType.DMA,
            pltpu.SemaphoreType.DMA,
            pltpu.SemaphoreType.DMA,
        ],
    )
    out5 = run(orig_p, new_weight, idsT)
    return out5.transpose(2, 4, 0, 1, 3).reshape(B, L, D)
